# Initial kernel scaffold; baseline (speedup 1.0000x reference)
import jax
import jax.numpy as jnp
from jax.experimental import pallas as pl


def kernel(h, edge_index, W1, b1, Wg0, bg0, Wg1, bg1, W2, b2):
    raise NotImplementedError("write your pallas kernel here")


# speedup vs baseline: 4.7934x; 4.7934x over previous
import functools

import jax
import jax.numpy as jnp
from jax import lax
from jax.experimental import pallas as pl
from jax.experimental.pallas import tpu as pltpu
from jax.experimental.pallas import tpu_sc as plsc

N = 10000
E = 320000
H = 128
C = 16
EPS = 0.5

NP = 10112
NW = 32
B = 128
NB = 79
PER_W = NB * B
EP = NW * PER_W
RPT = NP // 16

_mesh = plsc.VectorSubcoreMesh(
    core_axis_name="c", subcore_axis_name="s", num_cores=2, num_subcores=16)

_f32 = jnp.float32


def _hist_body(rows_hbm, cols_hbm, deg_out, cnt_out, dacc, cacc, idx_v, ones_v):
    cid = lax.axis_index("c")
    sid = lax.axis_index("s")
    wid = cid * 16 + sid
    lane = lax.broadcasted_iota(jnp.int32, (16,), 0)
    one_row = jnp.where(lane == 0, 1.0, 0.0).astype(_f32)
    zero_row = jnp.zeros((16,), _f32)

    def _fill(i, carry):
        ones_v[i, :] = one_row
        return carry

    def _zrow(i, carry):
        ones_v[i, :] = zero_row
        return carry

    r0 = sid * RPT
    lax.fori_loop(0, B, _zrow, 0)
    for k in range(4):
        pltpu.sync_copy(ones_v, dacc.at[pl.ds(r0 + k * B, B)])
        pltpu.sync_copy(ones_v, cacc.at[pl.ds(r0 + k * B, B)])
    pltpu.sync_copy(ones_v.at[pl.ds(0, RPT - 4 * B)],
                    dacc.at[pl.ds(r0 + 4 * B, RPT - 4 * B)])
    pltpu.sync_copy(ones_v.at[pl.ds(0, RPT - 4 * B)],
                    cacc.at[pl.ds(r0 + 4 * B, RPT - 4 * B)])
    lax.fori_loop(0, B, _fill, 0)
    plsc.subcore_barrier()

    def _blk(i, carry):
        base = wid * PER_W + i * B
        pltpu.sync_copy(rows_hbm.at[pl.ds(base, B)], idx_v.at[0])
        pltpu.sync_copy(cols_hbm.at[pl.ds(base, B)], idx_v.at[1])
        pltpu.sync_copy(ones_v, dacc.at[idx_v.at[0]], add=True)
        pltpu.sync_copy(ones_v, cacc.at[idx_v.at[1]], add=True)
        return carry
    lax.fori_loop(0, NB, _blk, 0)
    plsc.subcore_barrier()

    pltpu.sync_copy(dacc.at[pl.ds(r0, RPT)], deg_out.at[cid, pl.ds(r0, RPT)])
    pltpu.sync_copy(cacc.at[pl.ds(r0, RPT)], cnt_out.at[cid, pl.ds(r0, RPT)])


_hist = functools.partial(
    pl.kernel,
    out_type=(jax.ShapeDtypeStruct((2, NP, 16), _f32),
              jax.ShapeDtypeStruct((2, NP, 16), _f32)),
    mesh=_mesh,
    compiler_params=pltpu.CompilerParams(
        needs_layout_passes=False, use_tc_tiling_on_sc=False),
    scratch_types=[
        pltpu.MemorySpace.VMEM_SHARED((NP, 16), _f32),
        pltpu.MemorySpace.VMEM_SHARED((NP, 16), _f32),
        pltpu.VMEM((2, B), jnp.int32),
        pltpu.VMEM((B, 16), _f32),
    ],
)(_hist_body)


def _edge_body(rows_hbm, cols_hbm, table_hbm, ur_hbm, uc_hbm,
               part_out, s_out, accp, accs, idx_v, gbuf, urb, ucb, sop):
    cid = lax.axis_index("c")
    sid = lax.axis_index("s")
    wid = cid * 16 + sid
    lane = lax.broadcasted_iota(jnp.int32, (16,), 0)
    e0_row = jnp.where(lane == 0, 1.0, 0.0).astype(_f32)
    zero_row = jnp.zeros((16,), _f32)

    def _zrow(i, carry):
        for j in range(8):
            gbuf[i, pl.ds(j * 16, 16)] = zero_row
        sop[i, :] = zero_row
        return carry
    lax.fori_loop(0, B, _zrow, 0)
    r0 = sid * RPT
    for k in range(4):
        pltpu.sync_copy(gbuf, accp.at[pl.ds(r0 + k * B, B)])
        pltpu.sync_copy(sop, accs.at[pl.ds(r0 + k * B, B)])
    pltpu.sync_copy(gbuf.at[pl.ds(0, RPT - 4 * B)],
                    accp.at[pl.ds(r0 + 4 * B, RPT - 4 * B)])
    pltpu.sync_copy(sop.at[pl.ds(0, RPT - 4 * B)],
                    accs.at[pl.ds(r0 + 4 * B, RPT - 4 * B)])
    plsc.subcore_barrier()

    def _blk(i, carry):
        base = wid * PER_W + i * B
        pltpu.sync_copy(rows_hbm.at[pl.ds(base, B)], idx_v.at[0])
        pltpu.sync_copy(cols_hbm.at[pl.ds(base, B)], idx_v.at[1])
        pltpu.sync_copy(table_hbm.at[idx_v.at[0]], gbuf)
        pltpu.sync_copy(ur_hbm.at[idx_v.at[0]], urb)
        pltpu.sync_copy(uc_hbm.at[idx_v.at[1]], ucb)

        def _scale(e, carry2):
            rv = urb[e, pl.ds(0, 16)]
            cv = ucb[e, pl.ds(0, 16)]
            xv = rv + cv
            ex = jnp.exp(-2.0 * jnp.abs(xv))
            thv = jnp.sign(xv) * (1.0 - ex) / (1.0 + ex)
            dv = rv * cv
            cf = thv[0] * dv[1]
            for j in range(8):
                gbuf[e, pl.ds(j * 16, 16)] = gbuf[e, pl.ds(j * 16, 16)] * cf
            sop[e, :] = e0_row * cf
            return carry2
        lax.fori_loop(0, B, _scale, 0)

        pltpu.sync_copy(gbuf, accp.at[idx_v.at[1]], add=True)
        pltpu.sync_copy(sop, accs.at[idx_v.at[1]], add=True)
        return carry
    lax.fori_loop(0, NB, _blk, 0)
    plsc.subcore_barrier()

    pltpu.sync_copy(accp.at[pl.ds(r0, RPT)], part_out.at[cid, pl.ds(r0, RPT)])
    pltpu.sync_copy(accs.at[pl.ds(r0, RPT)], s_out.at[cid, pl.ds(r0, RPT)])


_edge = functools.partial(
    pl.kernel,
    out_type=(jax.ShapeDtypeStruct((2, NP, H), _f32),
              jax.ShapeDtypeStruct((2, NP, 16), _f32)),
    mesh=_mesh,
    compiler_params=pltpu.CompilerParams(
        needs_layout_passes=False, use_tc_tiling_on_sc=False),
    scratch_types=[
        pltpu.MemorySpace.VMEM_SHARED((NP, H), _f32),
        pltpu.MemorySpace.VMEM_SHARED((NP, 16), _f32),
        pltpu.VMEM((2, B), jnp.int32),
        pltpu.VMEM((B, H), _f32),
        pltpu.VMEM((B, 16), _f32),
        pltpu.VMEM((B, 16), _f32),
        pltpu.VMEM((B, 16), _f32),
    ],
)(_edge_body)


_GRID = 10
_BR = N // _GRID


def _mm0_body(h_ref, W1_ref, b1_ref, Wg_ref, bgv_ref, hh_ref, g_ref):
    hh = jnp.maximum(
        jnp.dot(h_ref[...], W1_ref[...], preferred_element_type=_f32)
        + b1_ref[...], 0.0)
    hh_ref[...] = hh
    g_ref[...] = jnp.dot(hh, Wg_ref[...], preferred_element_type=_f32) + bgv_ref[...]


def _tc_mm0(h, W1, b1r, Wgcat, bgv):
    return pl.pallas_call(
        _mm0_body,
        grid=(_GRID,),
        in_specs=[
            pl.BlockSpec((_BR, H), lambda i: (i, 0)),
            pl.BlockSpec((H, H), lambda i: (0, 0)),
            pl.BlockSpec((1, H), lambda i: (0, 0)),
            pl.BlockSpec((H, 2), lambda i: (0, 0)),
            pl.BlockSpec((1, 2), lambda i: (0, 0)),
        ],
        out_specs=[
            pl.BlockSpec((_BR, H), lambda i: (i, 0)),
            pl.BlockSpec((_BR, 2), lambda i: (i, 0)),
        ],
        out_shape=[
            jax.ShapeDtypeStruct((N, H), _f32),
            jax.ShapeDtypeStruct((N, 2), _f32),
        ],
    )(h, W1, b1r, Wgcat, bgv)


def _prep_body(d_ref, c_ref, dinv_ref, cnti_ref):
    deg = d_ref[0, :, 0:1] + d_ref[1, :, 0:1]
    cnt = c_ref[0, :, 0:1] + c_ref[1, :, 0:1]
    rowid = lax.broadcasted_iota(jnp.int32, (NP, 1), 0)
    dinv_ref[...] = jnp.where(
        rowid < N, lax.rsqrt(jnp.maximum(deg, 1.0)), 0.0)
    cnti_ref[...] = 1.0 / jnp.maximum(cnt, 1.0)


def _tc_prep(degs, cnts):
    return pl.pallas_call(
        _prep_body,
        out_shape=[
            jax.ShapeDtypeStruct((NP, 1), _f32),
            jax.ShapeDtypeStruct((NP, 1), _f32),
        ],
    )(degs, cnts)


def _comb_body(lin_ref, raw_ref, p_ref, s_ref, ci_ref, Wg_ref, bgv_ref,
               hh2_ref, g_ref):
    P = p_ref[0] + p_ref[1]
    S = s_ref[0, :, 0:1] + s_ref[1, :, 0:1]
    out = (S * lin_ref[...] + P) * ci_ref[...]
    hh2 = EPS * raw_ref[...] + out
    hh2_ref[...] = hh2
    g_ref[...] = jnp.dot(hh2, Wg_ref[...], preferred_element_type=_f32) + bgv_ref[...]


def _tc_comb(layer_in, raw, part, svec, cntinv, Wgcat, bgv):
    return pl.pallas_call(
        _comb_body,
        grid=(_GRID,),
        in_specs=[
            pl.BlockSpec((_BR, H), lambda i: (i, 0)),
            pl.BlockSpec((_BR, H), lambda i: (i, 0)),
            pl.BlockSpec((2, _BR, H), lambda i: (0, i, 0)),
            pl.BlockSpec((2, _BR, 16), lambda i: (0, i, 0)),
            pl.BlockSpec((_BR, 1), lambda i: (i, 0)),
            pl.BlockSpec((H, 2), lambda i: (0, 0)),
            pl.BlockSpec((1, 2), lambda i: (0, 0)),
        ],
        out_specs=[
            pl.BlockSpec((_BR, H), lambda i: (i, 0)),
            pl.BlockSpec((_BR, 2), lambda i: (i, 0)),
        ],
        out_shape=[
            jax.ShapeDtypeStruct((N, H), _f32),
            jax.ShapeDtypeStruct((N, 2), _f32),
        ],
    )(layer_in, raw, part, svec, cntinv, Wgcat, bgv)


def _final_body(lin_ref, raw_ref, p_ref, s_ref, ci_ref, W2_ref, b2_ref,
                out_ref):
    P = p_ref[0] + p_ref[1]
    S = s_ref[0, :, 0:1] + s_ref[1, :, 0:1]
    out = (S * lin_ref[...] + P) * ci_ref[...]
    hh3 = EPS * raw_ref[...] + out
    logits = jnp.dot(hh3, W2_ref[...], preferred_element_type=_f32) + b2_ref[...]
    m = jnp.max(logits, axis=1, keepdims=True)
    lse = m + jnp.log(jnp.sum(jnp.exp(logits - m), axis=1, keepdims=True))
    out_ref[...] = logits - lse


def _tc_final(layer_in, raw, part, svec, cntinv, W2, b2r):
    return pl.pallas_call(
        _final_body,
        grid=(_GRID,),
        in_specs=[
            pl.BlockSpec((_BR, H), lambda i: (i, 0)),
            pl.BlockSpec((_BR, H), lambda i: (i, 0)),
            pl.BlockSpec((2, _BR, H), lambda i: (0, i, 0)),
            pl.BlockSpec((2, _BR, 16), lambda i: (0, i, 0)),
            pl.BlockSpec((_BR, 1), lambda i: (i, 0)),
            pl.BlockSpec((H, C), lambda i: (0, 0)),
            pl.BlockSpec((1, C), lambda i: (0, 0)),
        ],
        out_specs=pl.BlockSpec((_BR, C), lambda i: (i, 0)),
        out_shape=jax.ShapeDtypeStruct((N, C), _f32),
    )(layer_in, raw, part, svec, cntinv, W2, b2r)


def kernel(h, edge_index, W1, b1, Wg0, bg0, Wg1, bg1, W2, b2):
    rows = jnp.pad(edge_index[0], (0, EP - E), constant_values=N)
    cols = jnp.pad(edge_index[1], (0, EP - E), constant_values=N)

    degs, cnts = _hist(rows, cols)
    dinv2, cntinv2 = _tc_prep(degs, cnts)
    dinv = dinv2.reshape(NP)
    cntinv = cntinv2[:N]

    Wg0cat = jnp.concatenate([Wg0[:H], Wg0[H:]], axis=1)
    Wg1cat = jnp.concatenate([Wg1[:H], Wg1[H:]], axis=1)
    bgv0 = jnp.stack([jnp.zeros((), _f32), bg0[0]]).reshape(1, 2)
    bgv1 = jnp.stack([jnp.zeros((), _f32), bg1[0]]).reshape(1, 2)

    hh, g0 = _tc_mm0(h, W1, b1.reshape(1, H), Wg0cat, bgv0)

    padn = ((0, NP - N), (0, 0))
    z14 = jnp.zeros((NP, 14), _f32)

    def _sidetables(g):
        a = jnp.pad(g[:, 0], (0, NP - N))
        b = jnp.pad(g[:, 1], (0, NP - N))
        ur = jnp.concatenate([jnp.stack([a, dinv], axis=1), z14], axis=1)
        uc = jnp.concatenate([jnp.stack([b, dinv], axis=1), z14], axis=1)
        return ur, uc

    table1 = jnp.pad(hh, padn)
    ur0, uc0 = _sidetables(g0)
    part1, s1 = _edge(rows, cols, table1, ur0, uc0)

    hh2, g1 = _tc_comb(hh, hh, part1[:, :N], s1[:, :N], cntinv, Wg1cat, bgv1)

    table2 = jnp.pad(hh2, padn)
    ur1, uc1 = _sidetables(g1)
    part2, s2 = _edge(rows, cols, table2, ur1, uc1)

    return _tc_final(hh2, hh, part2[:, :N], s2[:, :N], cntinv, W2,
                     b2.reshape(1, C))


# speedup vs baseline: 6.2085x; 1.2952x over previous
import functools

import jax
import jax.numpy as jnp
from jax import lax
from jax.experimental import pallas as pl
from jax.experimental.pallas import tpu as pltpu
from jax.experimental.pallas import tpu_sc as plsc

N = 10000
E = 320000
H = 128
C = 16
EPS = 0.5

NP = 10112
NW = 32
B = 128
NB = 79
PER_W = NB * B
EP = NW * PER_W
RPT = NP // 16

_mesh = plsc.VectorSubcoreMesh(
    core_axis_name="c", subcore_axis_name="s", num_cores=2, num_subcores=16)

_f32 = jnp.float32


def _hist_body(rows_hbm, cols_hbm, deg_out, cnt_out, dacc, cacc, idx_v, ones_v):
    cid = lax.axis_index("c")
    sid = lax.axis_index("s")
    wid = cid * 16 + sid
    lane = lax.broadcasted_iota(jnp.int32, (16,), 0)
    one_row = jnp.where(lane == 0, 1.0, 0.0).astype(_f32)
    zero_row = jnp.zeros((16,), _f32)

    def _fill(i, carry):
        ones_v[i, :] = one_row
        return carry

    def _zrow(i, carry):
        ones_v[i, :] = zero_row
        return carry

    r0 = sid * RPT
    lax.fori_loop(0, B, _zrow, 0)
    for k in range(4):
        pltpu.sync_copy(ones_v, dacc.at[pl.ds(r0 + k * B, B)])
        pltpu.sync_copy(ones_v, cacc.at[pl.ds(r0 + k * B, B)])
    pltpu.sync_copy(ones_v.at[pl.ds(0, RPT - 4 * B)],
                    dacc.at[pl.ds(r0 + 4 * B, RPT - 4 * B)])
    pltpu.sync_copy(ones_v.at[pl.ds(0, RPT - 4 * B)],
                    cacc.at[pl.ds(r0 + 4 * B, RPT - 4 * B)])
    lax.fori_loop(0, B, _fill, 0)
    plsc.subcore_barrier()

    def _blk(i, carry):
        base = wid * PER_W + i * B
        pltpu.sync_copy(rows_hbm.at[pl.ds(base, B)], idx_v.at[0])
        pltpu.sync_copy(cols_hbm.at[pl.ds(base, B)], idx_v.at[1])
        pltpu.sync_copy(ones_v, dacc.at[idx_v.at[0]], add=True)
        pltpu.sync_copy(ones_v, cacc.at[idx_v.at[1]], add=True)
        return carry
    lax.fori_loop(0, NB, _blk, 0)
    plsc.subcore_barrier()

    pltpu.sync_copy(dacc.at[pl.ds(r0, RPT)], deg_out.at[cid, pl.ds(r0, RPT)])
    pltpu.sync_copy(cacc.at[pl.ds(r0, RPT)], cnt_out.at[cid, pl.ds(r0, RPT)])


_hist = functools.partial(
    pl.kernel,
    out_type=(jax.ShapeDtypeStruct((2, NP, 16), _f32),
              jax.ShapeDtypeStruct((2, NP, 16), _f32)),
    mesh=_mesh,
    compiler_params=pltpu.CompilerParams(
        needs_layout_passes=False, use_tc_tiling_on_sc=False),
    scratch_types=[
        pltpu.MemorySpace.VMEM_SHARED((NP, 16), _f32),
        pltpu.MemorySpace.VMEM_SHARED((NP, 16), _f32),
        pltpu.VMEM((2, B), jnp.int32),
        pltpu.VMEM((B, 16), _f32),
    ],
)(_hist_body)


def _edge_body(rows_hbm, cols_hbm, table_hbm, ur_hbm, uc_hbm,
               part_out, s_out, accp, accs, idx_v, gbuf, urb, ucb, sop,
               coef_v):
    cid = lax.axis_index("c")
    sid = lax.axis_index("s")
    wid = cid * 16 + sid
    lane = lax.broadcasted_iota(jnp.int32, (16,), 0)
    e0_row = jnp.where(lane == 0, 1.0, 0.0).astype(_f32)
    zero_row = jnp.zeros((16,), _f32)

    def _zrow(i, carry):
        for j in range(8):
            gbuf[i, pl.ds(j * 16, 16)] = zero_row
        sop[i, :] = zero_row
        return carry
    lax.fori_loop(0, B, _zrow, 0)
    r0 = sid * RPT
    for k in range(4):
        pltpu.sync_copy(gbuf, accp.at[pl.ds(r0 + k * B, B)])
        pltpu.sync_copy(sop, accs.at[pl.ds(r0 + k * B, B)])
    pltpu.sync_copy(gbuf.at[pl.ds(0, RPT - 4 * B)],
                    accp.at[pl.ds(r0 + 4 * B, RPT - 4 * B)])
    pltpu.sync_copy(sop.at[pl.ds(0, RPT - 4 * B)],
                    accs.at[pl.ds(r0 + 4 * B, RPT - 4 * B)])
    plsc.subcore_barrier()

    def _blk(i, carry):
        base = wid * PER_W + i * B
        pltpu.sync_copy(rows_hbm.at[pl.ds(base, B)], idx_v.at[0])
        pltpu.sync_copy(cols_hbm.at[pl.ds(base, B)], idx_v.at[1])
        pltpu.sync_copy(table_hbm.at[idx_v.at[0]], gbuf)
        pltpu.sync_copy(ur_hbm.at[idx_v.at[0]], urb)
        pltpu.sync_copy(uc_hbm.at[idx_v.at[1]], ucb)

        z16 = lane * 0

        def _coef(g, carry2):
            ridx = g * 16 + lane
            ar = plsc.load_gather(urb, [ridx, z16])
            dr = plsc.load_gather(urb, [ridx, z16 + 1])
            bc = plsc.load_gather(ucb, [ridx, z16])
            dc = plsc.load_gather(ucb, [ridx, z16 + 1])
            xv = ar + bc
            ex = jnp.exp(-2.0 * jnp.abs(xv))
            thv = jnp.sign(xv) * (1.0 - ex) / (1.0 + ex)
            coef_v[pl.ds(g * 16, 16)] = thv * dr * dc
            return carry2
        lax.fori_loop(0, B // 16, _coef, 0)

        def _scale(e, carry2):
            cf = coef_v[pl.ds(e, 16)][0]
            for j in range(8):
                gbuf[e, pl.ds(j * 16, 16)] = gbuf[e, pl.ds(j * 16, 16)] * cf
            sop[e, :] = e0_row * cf
            return carry2
        lax.fori_loop(0, B, _scale, 0)

        pltpu.sync_copy(gbuf, accp.at[idx_v.at[1]], add=True)
        pltpu.sync_copy(sop, accs.at[idx_v.at[1]], add=True)
        return carry
    lax.fori_loop(0, NB, _blk, 0)
    plsc.subcore_barrier()

    pltpu.sync_copy(accp.at[pl.ds(r0, RPT)], part_out.at[cid, pl.ds(r0, RPT)])
    pltpu.sync_copy(accs.at[pl.ds(r0, RPT)], s_out.at[cid, pl.ds(r0, RPT)])


_edge = functools.partial(
    pl.kernel,
    out_type=(jax.ShapeDtypeStruct((2, NP, H), _f32),
              jax.ShapeDtypeStruct((2, NP, 16), _f32)),
    mesh=_mesh,
    compiler_params=pltpu.CompilerParams(
        needs_layout_passes=False, use_tc_tiling_on_sc=False),
    scratch_types=[
        pltpu.MemorySpace.VMEM_SHARED((NP, H), _f32),
        pltpu.MemorySpace.VMEM_SHARED((NP, 16), _f32),
        pltpu.VMEM((2, B), jnp.int32),
        pltpu.VMEM((B, H), _f32),
        pltpu.VMEM((B, 16), _f32),
        pltpu.VMEM((B, 16), _f32),
        pltpu.VMEM((B, 16), _f32),
        pltpu.VMEM((B + 16,), _f32),
    ],
)(_edge_body)


_GRID = 10
_BR = N // _GRID


def _mm0_body(h_ref, W1_ref, b1_ref, Wg_ref, bgv_ref, hh_ref, g_ref):
    hh = jnp.maximum(
        jnp.dot(h_ref[...], W1_ref[...], preferred_element_type=_f32)
        + b1_ref[...], 0.0)
    hh_ref[...] = hh
    g_ref[...] = jnp.dot(hh, Wg_ref[...], preferred_element_type=_f32) + bgv_ref[...]


def _tc_mm0(h, W1, b1r, Wgcat, bgv):
    return pl.pallas_call(
        _mm0_body,
        grid=(_GRID,),
        in_specs=[
            pl.BlockSpec((_BR, H), lambda i: (i, 0)),
            pl.BlockSpec((H, H), lambda i: (0, 0)),
            pl.BlockSpec((1, H), lambda i: (0, 0)),
            pl.BlockSpec((H, 2), lambda i: (0, 0)),
            pl.BlockSpec((1, 2), lambda i: (0, 0)),
        ],
        out_specs=[
            pl.BlockSpec((_BR, H), lambda i: (i, 0)),
            pl.BlockSpec((_BR, 2), lambda i: (i, 0)),
        ],
        out_shape=[
            jax.ShapeDtypeStruct((N, H), _f32),
            jax.ShapeDtypeStruct((N, 2), _f32),
        ],
    )(h, W1, b1r, Wgcat, bgv)


def _prep_body(d_ref, c_ref, dinv_ref, cnti_ref):
    deg = d_ref[0, :, 0:1] + d_ref[1, :, 0:1]
    cnt = c_ref[0, :, 0:1] + c_ref[1, :, 0:1]
    rowid = lax.broadcasted_iota(jnp.int32, (NP, 1), 0)
    dinv_ref[...] = jnp.where(
        rowid < N, lax.rsqrt(jnp.maximum(deg, 1.0)), 0.0)
    cnti_ref[...] = 1.0 / jnp.maximum(cnt, 1.0)


def _tc_prep(degs, cnts):
    return pl.pallas_call(
        _prep_body,
        out_shape=[
            jax.ShapeDtypeStruct((NP, 1), _f32),
            jax.ShapeDtypeStruct((NP, 1), _f32),
        ],
    )(degs, cnts)


def _comb_body(lin_ref, raw_ref, p_ref, s_ref, ci_ref, Wg_ref, bgv_ref,
               hh2_ref, g_ref):
    P = p_ref[0] + p_ref[1]
    S = s_ref[0, :, 0:1] + s_ref[1, :, 0:1]
    out = (S * lin_ref[...] + P) * ci_ref[...]
    hh2 = EPS * raw_ref[...] + out
    hh2_ref[...] = hh2
    g_ref[...] = jnp.dot(hh2, Wg_ref[...], preferred_element_type=_f32) + bgv_ref[...]


def _tc_comb(layer_in, raw, part, svec, cntinv, Wgcat, bgv):
    return pl.pallas_call(
        _comb_body,
        grid=(_GRID,),
        in_specs=[
            pl.BlockSpec((_BR, H), lambda i: (i, 0)),
            pl.BlockSpec((_BR, H), lambda i: (i, 0)),
            pl.BlockSpec((2, _BR, H), lambda i: (0, i, 0)),
            pl.BlockSpec((2, _BR, 16), lambda i: (0, i, 0)),
            pl.BlockSpec((_BR, 1), lambda i: (i, 0)),
            pl.BlockSpec((H, 2), lambda i: (0, 0)),
            pl.BlockSpec((1, 2), lambda i: (0, 0)),
        ],
        out_specs=[
            pl.BlockSpec((_BR, H), lambda i: (i, 0)),
            pl.BlockSpec((_BR, 2), lambda i: (i, 0)),
        ],
        out_shape=[
            jax.ShapeDtypeStruct((N, H), _f32),
            jax.ShapeDtypeStruct((N, 2), _f32),
        ],
    )(layer_in, raw, part, svec, cntinv, Wgcat, bgv)


def _final_body(lin_ref, raw_ref, p_ref, s_ref, ci_ref, W2_ref, b2_ref,
                out_ref):
    P = p_ref[0] + p_ref[1]
    S = s_ref[0, :, 0:1] + s_ref[1, :, 0:1]
    out = (S * lin_ref[...] + P) * ci_ref[...]
    hh3 = EPS * raw_ref[...] + out
    logits = jnp.dot(hh3, W2_ref[...], preferred_element_type=_f32) + b2_ref[...]
    m = jnp.max(logits, axis=1, keepdims=True)
    lse = m + jnp.log(jnp.sum(jnp.exp(logits - m), axis=1, keepdims=True))
    out_ref[...] = logits - lse


def _tc_final(layer_in, raw, part, svec, cntinv, W2, b2r):
    return pl.pallas_call(
        _final_body,
        grid=(_GRID,),
        in_specs=[
            pl.BlockSpec((_BR, H), lambda i: (i, 0)),
            pl.BlockSpec((_BR, H), lambda i: (i, 0)),
            pl.BlockSpec((2, _BR, H), lambda i: (0, i, 0)),
            pl.BlockSpec((2, _BR, 16), lambda i: (0, i, 0)),
            pl.BlockSpec((_BR, 1), lambda i: (i, 0)),
            pl.BlockSpec((H, C), lambda i: (0, 0)),
            pl.BlockSpec((1, C), lambda i: (0, 0)),
        ],
        out_specs=pl.BlockSpec((_BR, C), lambda i: (i, 0)),
        out_shape=jax.ShapeDtypeStruct((N, C), _f32),
    )(layer_in, raw, part, svec, cntinv, W2, b2r)


def kernel(h, edge_index, W1, b1, Wg0, bg0, Wg1, bg1, W2, b2):
    rows = jnp.pad(edge_index[0], (0, EP - E), constant_values=N)
    cols = jnp.pad(edge_index[1], (0, EP - E), constant_values=N)

    degs, cnts = _hist(rows, cols)
    dinv2, cntinv2 = _tc_prep(degs, cnts)
    dinv = dinv2.reshape(NP)
    cntinv = cntinv2[:N]

    Wg0cat = jnp.concatenate([Wg0[:H], Wg0[H:]], axis=1)
    Wg1cat = jnp.concatenate([Wg1[:H], Wg1[H:]], axis=1)
    bgv0 = jnp.stack([jnp.zeros((), _f32), bg0[0]]).reshape(1, 2)
    bgv1 = jnp.stack([jnp.zeros((), _f32), bg1[0]]).reshape(1, 2)

    hh, g0 = _tc_mm0(h, W1, b1.reshape(1, H), Wg0cat, bgv0)

    padn = ((0, NP - N), (0, 0))
    z14 = jnp.zeros((NP, 14), _f32)

    def _sidetables(g):
        a = jnp.pad(g[:, 0], (0, NP - N))
        b = jnp.pad(g[:, 1], (0, NP - N))
        ur = jnp.concatenate([jnp.stack([a, dinv], axis=1), z14], axis=1)
        uc = jnp.concatenate([jnp.stack([b, dinv], axis=1), z14], axis=1)
        return ur, uc

    table1 = jnp.pad(hh, padn)
    ur0, uc0 = _sidetables(g0)
    part1, s1 = _edge(rows, cols, table1, ur0, uc0)

    hh2, g1 = _tc_comb(hh, hh, part1[:, :N], s1[:, :N], cntinv, Wg1cat, bgv1)

    table2 = jnp.pad(hh2, padn)
    ur1, uc1 = _sidetables(g1)
    part2, s2 = _edge(rows, cols, table2, ur1, uc1)

    return _tc_final(hh2, hh, part2[:, :N], s2[:, :N], cntinv, W2,
                     b2.reshape(1, C))
